# baseline (speedup 1.0000x reference)
import functools

import jax
import jax.numpy as jnp
from jax import lax
from jax.experimental import pallas as pl
from jax.experimental.pallas import tpu as pltpu
from jax.experimental.pallas import tpu_sc as plsc

N = 10000
NPAD = 10240
E = 320000
EPAD = 327680
ROWS_PER_TILE = NPAD // 16
SIGMA = 0.5
NB = 4


def _make_spmm(dsc, ce, col_split):
  nchunks = EPAD // ce
  nstage = nchunks // 16 if col_split else nchunks // 32
  mesh = plsc.VectorSubcoreMesh(core_axis_name="c", subcore_axis_name="s",
                                num_cores=2, num_subcores=16)

  @functools.partial(
      pl.kernel,
      out_type=jax.ShapeDtypeStruct((2 * NPAD, dsc), jnp.float32),
      mesh=mesh,
      scratch_types=[
          pltpu.VMEM((nstage, ce), jnp.int32),
          [pltpu.VMEM((ce, dsc), jnp.float32)] * NB,
          [pltpu.VMEM((ce,), jnp.float32)] * NB,
          [pltpu.VMEM((ce,), jnp.int32)] * NB,
          pltpu.VMEM_SHARED((NPAD, dsc), jnp.float32),
          [pltpu.SemaphoreType.DMA] * NB,
          [pltpu.SemaphoreType.DMA] * NB,
      ],
      compiler_params=pltpu.CompilerParams(use_tc_tiling_on_sc=False),
  )
  def spmm(x_hbm, src_hbm, dst_hbm, w_hbm, out_hbm,
           src_v, bufs, wslots, dslots, acc_s, gsems, ssems):
    c = lax.axis_index("c")
    s = lax.axis_index("s")
    ebase = (s * 2 + c) * nstage
    base = s * ROWS_PER_TILE

    r0 = bufs[0]
    zero16 = jnp.zeros((16,), jnp.float32)

    def zrow(i, carry):
      for cb in range(dsc // 16):
        r0[i, pl.ds(cb * 16, 16)] = zero16
      return carry

    lax.fori_loop(0, ce, zrow, 0, unroll=2)

    for r in range(ROWS_PER_TILE // ce):
      pltpu.sync_copy(r0, acc_s.at[pl.ds(base + r * ce, ce)])

    pltpu.sync_copy(src_hbm.at[pl.ds(ebase, nstage)], src_v)

    plsc.subcore_barrier()

    def gather(j, b):
      pltpu.async_copy(x_hbm.at[src_v.at[j]], bufs[b], gsems[b])
      off = (ebase + j) * ce
      pltpu.async_copy(w_hbm.at[pl.ds(off, ce)], wslots[b], gsems[b])
      pltpu.async_copy(dst_hbm.at[pl.ds(off, ce)], dslots[b], gsems[b])

    def gwait(b):
      pltpu.make_async_copy(x_hbm.at[pl.ds(0, ce)], bufs[b], gsems[b]).wait()
      pltpu.make_async_copy(w_hbm.at[pl.ds(0, ce)], wslots[b], gsems[b]).wait()
      pltpu.make_async_copy(dst_hbm.at[pl.ds(0, ce)], dslots[b],
                            gsems[b]).wait()

    def scatter(j, b):
      pltpu.async_copy(bufs[b], acc_s.at[dslots[b]], ssems[b], add=True)

    def swait(b):
      pltpu.make_async_copy(x_hbm.at[pl.ds(0, ce)], bufs[b], ssems[b]).wait()

    def mult(j, b):
      buf = bufs[b]
      wsl = wslots[b]

      def grp_body(g, cc):
        wvec = wsl[pl.ds(g * 16, 16)]
        for l in range(16):
          wl = wvec[l]
          e = g * 16 + l
          for cb in range(dsc // 16):
            sl = pl.ds(cb * 16, 16)
            buf[e, sl] = buf[e, sl] * wl
        return cc

      lax.fori_loop(0, ce // 16, grp_body, 0)

    gather(0, 0)
    gather(1, 1)
    for b in range(NB):
      gwait(b)
      mult(b, b)
      scatter(b, b)
      b2 = (b + 2) % NB
      if b >= NB - 2:
        swait(b2)
      gather(b + 2, b2)

    def block_body(g, carry):
      for b in range(NB):
        j = g * NB + b
        gwait(b)
        mult(j, b)
        scatter(j, b)
        b2 = (b + 2) % NB
        swait(b2)
        gather(j + 2, b2)
      return carry

    lax.fori_loop(1, nstage // NB - 1, block_body, 0)

    for b in range(NB):
      j = nstage - NB + b
      gwait(b)
      mult(j, b)
      scatter(j, b)
      if b < NB - 2:
        b2 = (b + 2) % NB
        swait(b2)
        gather(j + 2, b2)
    for b in range(NB):
      swait(b)

    plsc.subcore_barrier()

    pltpu.sync_copy(acc_s.at[pl.ds(base, ROWS_PER_TILE)],
                    out_hbm.at[pl.ds(c * NPAD + base, ROWS_PER_TILE)])

  return spmm


_spmm_cache = {}


def _spmm(dsc, ce, col_split):
  key = (dsc, ce, col_split)
  if key not in _spmm_cache:
    _spmm_cache[key] = _make_spmm(dsc, ce, col_split)
  return _spmm_cache[key]


def _bn(t, g, b):
  m = jnp.mean(t, axis=0, keepdims=True)
  v = jnp.mean(jnp.square(t - m), axis=0, keepdims=True)
  return g * (t - m) / jnp.sqrt(v + 1e-5) + b


def _psum(p_ref):
  return p_ref[pl.ds(0, N), :] + p_ref[pl.ds(NPAD, N), :]


def _tc_mid_body(p_ref, w_ref, g_ref, b_ref, tra_ref, o_ref):
  t = jnp.dot(_psum(p_ref), w_ref[...], preferred_element_type=jnp.float32)
  h = jnp.maximum(_bn(t, g_ref[...], b_ref[...]), 0.0)
  o_ref[...] = (1.0 - SIGMA) * h + SIGMA * tra_ref[...]


def _tc_mid(p, w, g, b, tra):
  return pl.pallas_call(
      _tc_mid_body,
      out_shape=jax.ShapeDtypeStruct((N, 128), jnp.float32),
  )(p, w, g, b, tra)


def _tc_l3_body(p_ref, w_ref, g_ref, b_ref, tra_ref, w4_ref, o_ref):
  t = jnp.dot(_psum(p_ref), w_ref[...], preferred_element_type=jnp.float32)
  h = jnp.maximum(_bn(t, g_ref[...], b_ref[...]), 0.0)
  u = (1.0 - SIGMA) * h + SIGMA * tra_ref[...]
  o_ref[...] = jnp.dot(u, w4_ref[...], preferred_element_type=jnp.float32)


def _tc_l3(p, w, g, b, tra, w4p):
  return pl.pallas_call(
      _tc_l3_body,
      out_shape=jax.ShapeDtypeStruct((N, 16), jnp.float32),
  )(p, w, g, b, tra, w4p)


def _psum16(p_ref):
  return (p_ref[pl.ds(0, N), :] + p_ref[pl.ds(NPAD, N), :])[:, :10]


def _tc_l4_body(p_ref, g_ref, b_ref, z_ref, w5_ref, o_ref):
  h = jnp.maximum(_bn(_psum16(p_ref), g_ref[...], b_ref[...]), 0.0)
  u = (1.0 - SIGMA) * h + SIGMA * z_ref[...]
  o_ref[...] = jnp.dot(u, w5_ref[...], preferred_element_type=jnp.float32)


def _tc_l4(p, g, b, z, w5p):
  return pl.pallas_call(
      _tc_l4_body,
      out_shape=jax.ShapeDtypeStruct((N, 16), jnp.float32),
  )(p, g, b, z, w5p)


def _tc_final_body(p_ref, g_ref, b_ref, z_ref, cl_ref,
                   q_ref, pred_ref, h5_ref):
  h5 = _bn(_psum16(p_ref), g_ref[...], b_ref[...])
  h5_ref[...] = h5
  nrm = jnp.sqrt(jnp.sum(h5 * h5, axis=1, keepdims=True)) + 1e-12
  pred_ref[...] = jax.nn.softmax(h5 / nrm, axis=1)
  z = z_ref[...]
  cl = cl_ref[...]
  z2 = jnp.sum(z * z, axis=1, keepdims=True)
  c2 = jnp.sum(cl * cl, axis=1)
  zc = lax.dot_general(z, cl, (((1,), (1,)), ((), ())),
                       preferred_element_type=jnp.float32)
  d2 = z2 + c2[None, :] - 2.0 * zc
  q = 1.0 / (1.0 + d2)
  q_ref[...] = q / jnp.sum(q, axis=1, keepdims=True)


def _tc_final(p, g, b, z, cl):
  return pl.pallas_call(
      _tc_final_body,
      out_shape=(
          jax.ShapeDtypeStruct((N, 4), jnp.float32),
          jax.ShapeDtypeStruct((N, 10), jnp.float32),
          jax.ShapeDtypeStruct((N, 10), jnp.float32),
      ),
  )(p, g, b, z, cl)


def kernel(encoded_input_data, tra1, tra2, tra3, z, edge_index, edge_weight,
           W1, g1, b1, W2, g2, b2, W3, g3, b3, W4, g4, b4, W5, g5, b5,
           cluster):
  pad = EPAD - E
  src = jnp.concatenate([edge_index[0], jnp.zeros((pad,), jnp.int32)])
  dst = jnp.concatenate([edge_index[1], jnp.zeros((pad,), jnp.int32)])
  wgt = jnp.concatenate([edge_weight, jnp.zeros((pad,), jnp.float32)])
  src64 = src.reshape(EPAD // 64, 64)
  src128 = src.reshape(EPAD // 128, 128)

  w4p = jnp.pad(W4, ((0, 0), (0, 6)))
  w5p = jnp.pad(W5, ((0, 0), (0, 6)))

  spmm128 = _spmm(128, 64, False)
  spmm16 = _spmm(16, 128, False)

  p1 = spmm128(encoded_input_data, src64, dst, wgt)
  u2 = _tc_mid(p1, W1, g1, b1, tra1)
  p2 = spmm128(u2, src64, dst, wgt)
  u3 = _tc_mid(p2, W2, g2, b2, tra2)
  p3 = spmm128(u3, src64, dst, wgt)
  y4 = _tc_l3(p3, W3, g3, b3, tra3, w4p)
  p4 = spmm16(y4, src128, dst, wgt)
  y5 = _tc_l4(p4, g4, b4, z, w5p)
  p5 = spmm16(y5, src128, dst, wgt)
  q, pred, h5 = _tc_final(p5, g5, b5, z, cluster)
  return (q, pred, h5)


# speedup vs baseline: 1.5178x; 1.5178x over previous
import functools

import numpy as np

import jax
import jax.numpy as jnp
from jax import lax
from jax.experimental import pallas as pl
from jax.experimental.pallas import tpu as pltpu
from jax.experimental.pallas import tpu_sc as plsc

N = 10000
NPAD = 10240
E = 320000
EPAD = 327680
ROWS_PER_TILE = NPAD // 16
SIGMA = 0.5
NB = 4


def _make_spmm(dsc, ce, col_split):
  nchunks = EPAD // ce
  nstage = nchunks // 16 if col_split else nchunks // 32
  mesh = plsc.VectorSubcoreMesh(core_axis_name="c", subcore_axis_name="s",
                                num_cores=2, num_subcores=16)

  @functools.partial(
      pl.kernel,
      out_type=jax.ShapeDtypeStruct((2 * NPAD, dsc), jnp.float32),
      mesh=mesh,
      scratch_types=[
          pltpu.VMEM((nstage, ce), jnp.int32),
          [pltpu.VMEM((ce, dsc), jnp.float32)] * NB,
          [pltpu.VMEM((ce,), jnp.float32)] * NB,
          [pltpu.VMEM((ce,), jnp.int32)] * NB,
          pltpu.VMEM_SHARED((NPAD, dsc), jnp.float32),
          [pltpu.SemaphoreType.DMA] * NB,
          [pltpu.SemaphoreType.DMA] * NB,
      ],
      compiler_params=pltpu.CompilerParams(use_tc_tiling_on_sc=False),
  )
  def spmm(x_hbm, src_hbm, dst_hbm, w_hbm, out_hbm,
           src_v, bufs, wslots, dslots, acc_s, gsems, ssems):
    c = lax.axis_index("c")
    s = lax.axis_index("s")
    ebase = (s * 2 + c) * nstage
    base = s * ROWS_PER_TILE

    r0 = bufs[0]
    zero16 = jnp.zeros((16,), jnp.float32)

    def zrow(i, carry):
      for cb in range(dsc // 16):
        r0[i, pl.ds(cb * 16, 16)] = zero16
      return carry

    lax.fori_loop(0, ce, zrow, 0, unroll=2)

    for r in range(ROWS_PER_TILE // ce):
      pltpu.sync_copy(r0, acc_s.at[pl.ds(base + r * ce, ce)])

    pltpu.sync_copy(src_hbm.at[pl.ds(ebase, nstage)], src_v)

    plsc.subcore_barrier()

    def gather(j, b):
      pltpu.async_copy(x_hbm.at[src_v.at[j]], bufs[b], gsems[b])
      off = (ebase + j) * ce
      pltpu.async_copy(w_hbm.at[pl.ds(off, ce)], wslots[b], gsems[b])
      pltpu.async_copy(dst_hbm.at[pl.ds(off, ce)], dslots[b], gsems[b])

    def gwait(b):
      pltpu.make_async_copy(x_hbm.at[pl.ds(0, ce)], bufs[b], gsems[b]).wait()
      pltpu.make_async_copy(w_hbm.at[pl.ds(0, ce)], wslots[b], gsems[b]).wait()
      pltpu.make_async_copy(dst_hbm.at[pl.ds(0, ce)], dslots[b],
                            gsems[b]).wait()

    def scatter(j, b):
      pltpu.async_copy(bufs[b], acc_s.at[dslots[b]], ssems[b], add=True)

    def swait(b):
      pltpu.make_async_copy(x_hbm.at[pl.ds(0, ce)], bufs[b], ssems[b]).wait()

    def mult(j, b):
      buf = bufs[b]
      wsl = wslots[b]

      def grp_body(g, cc):
        wvec = wsl[pl.ds(g * 16, 16)]
        for l in range(16):
          wl = wvec[l]
          e = g * 16 + l
          for cb in range(dsc // 16):
            sl = pl.ds(cb * 16, 16)
            buf[e, sl] = buf[e, sl] * wl
        return cc

      lax.fori_loop(0, ce // 16, grp_body, 0)

    gather(0, 0)
    gather(1, 1)
    for b in range(NB):
      gwait(b)
      mult(b, b)
      scatter(b, b)
      b2 = (b + 2) % NB
      if b >= NB - 2:
        swait(b2)
      gather(b + 2, b2)

    def block_body(g, carry):
      for b in range(NB):
        j = g * NB + b
        gwait(b)
        mult(j, b)
        scatter(j, b)
        b2 = (b + 2) % NB
        swait(b2)
        gather(j + 2, b2)
      return carry

    lax.fori_loop(1, nstage // NB - 1, block_body, 0)

    for b in range(NB):
      j = nstage - NB + b
      gwait(b)
      mult(j, b)
      scatter(j, b)
      if b < NB - 2:
        b2 = (b + 2) % NB
        swait(b2)
        gather(j + 2, b2)
    for b in range(NB):
      swait(b)

    plsc.subcore_barrier()

    pltpu.sync_copy(acc_s.at[pl.ds(base, ROWS_PER_TILE)],
                    out_hbm.at[pl.ds(c * NPAD + base, ROWS_PER_TILE)])

  return spmm


def _make_spmm_bf16():
  ce = 64
  nstage = EPAD // ce // 32
  mesh = plsc.VectorSubcoreMesh(core_axis_name="c", subcore_axis_name="s",
                                num_cores=2, num_subcores=16)

  @functools.partial(
      pl.kernel,
      out_type=jax.ShapeDtypeStruct((2 * NPAD, 128), jnp.float32),
      mesh=mesh,
      scratch_types=[
          pltpu.VMEM((nstage, ce), jnp.int32),
          [pltpu.VMEM((ce, 64), jnp.int32)] * 4,
          [pltpu.VMEM((ce, 128), jnp.float32)] * 2,
          [pltpu.VMEM((ce,), jnp.float32)] * 4,
          [pltpu.VMEM((ce,), jnp.int32)] * 4,
          pltpu.VMEM_SHARED((NPAD, 128), jnp.float32),
          [pltpu.SemaphoreType.DMA] * 4,
          [pltpu.SemaphoreType.DMA] * 4,
          [pltpu.SemaphoreType.DMA] * 2,
      ],
      compiler_params=pltpu.CompilerParams(use_tc_tiling_on_sc=False,
                                     needs_layout_passes=False),
  )
  def spmm(x_hbm, src_hbm, dst_hbm, w_hbm, out_hbm,
           src_v, gbufs, fbufs, wslots, dslots, acc_s, gsems, esems, ssems):
    c = lax.axis_index("c")
    s = lax.axis_index("s")
    ebase = (s * 2 + c) * nstage
    base = s * ROWS_PER_TILE

    f0 = fbufs[0]
    zero16 = jnp.zeros((16,), jnp.float32)

    def zrow(i, carry):
      for cb in range(8):
        f0[i, pl.ds(cb * 16, 16)] = zero16
      return carry

    lax.fori_loop(0, ce, zrow, 0, unroll=2)

    for r in range(ROWS_PER_TILE // ce):
      pltpu.sync_copy(f0, acc_s.at[pl.ds(base + r * ce, ce)])

    pltpu.sync_copy(src_hbm.at[pl.ds(ebase, nstage)], src_v)

    plsc.subcore_barrier()

    def gather(j, b):
      pltpu.async_copy(x_hbm.at[src_v.at[j]], gbufs[b], gsems[b])

    def gwait(b):
      pltpu.make_async_copy(src_hbm.at[pl.ds(0, ce)], gbufs[b],
                            gsems[b]).wait()

    def load_wd(j, b):
      off = (ebase + j) * ce
      pltpu.async_copy(w_hbm.at[pl.ds(off, ce)], wslots[b], esems[b])
      pltpu.async_copy(dst_hbm.at[pl.ds(off, ce)], dslots[b], esems[b])

    def ewait(b):
      pltpu.make_async_copy(w_hbm.at[pl.ds(0, ce)], wslots[b],
                            esems[b]).wait()
      pltpu.make_async_copy(dst_hbm.at[pl.ds(0, ce)], dslots[b],
                            esems[b]).wait()

    def scatter(j, b4, b2):
      pltpu.async_copy(fbufs[b2], acc_s.at[dslots[b4]], ssems[b2], add=True)

    def swait(b2):
      pltpu.make_async_copy(out_hbm.at[pl.ds(0, ce)], fbufs[b2],
                            ssems[b2]).wait()

    himask = jnp.int32(-65536)

    def mult(j, b4, b2):
      gb = gbufs[b4]
      fbuf = fbufs[b2]
      wsl = wslots[b4]

      def grp_body(g, cc):
        wvec = wsl[pl.ds(g * 16, 16)]
        for l in range(16):
          wl = wvec[l]
          e = g * 16 + l
          for k in range(4):
            w32 = gb[e, pl.ds(k * 16, 16)]
            lo = plsc.bitcast(w32 << 16, jnp.float32)
            hi = plsc.bitcast(w32 & himask, jnp.float32)
            fbuf[e, pl.ds(k * 32, 16)] = lo * wl
            fbuf[e, pl.ds(k * 32 + 16, 16)] = hi * wl
        return cc

      lax.fori_loop(0, ce // 16, grp_body, 0)

    def body(j, b4, b2, with_gather, with_swait, with_wd):
      if with_gather:
        gather(j + 3, (b4 + 3) % 4)
      gwait(b4)
      if with_swait:
        swait(b2)
      if with_wd:
        load_wd(j + 2, (b4 + 2) % 4)
      ewait(b4)
      mult(j, b4, b2)
      scatter(j, b4, b2)

    gather(0, 0)
    gather(1, 1)
    gather(2, 2)
    load_wd(0, 0)
    load_wd(1, 1)
    for j in range(4):
      body(j, j % 4, j % 2, True, j >= 2, True)

    def block_body(g, carry):
      for b in range(4):
        body(g * 4 + b, b, b % 2, True, True, True)
      return carry

    lax.fori_loop(1, nstage // 4 - 1, block_body, 0)

    for b in range(4):
      j = nstage - 4 + b
      body(j, b, b % 2, j + 3 < nstage, True, j + 2 < nstage)
    swait(0)
    swait(1)

    plsc.subcore_barrier()

    pltpu.sync_copy(acc_s.at[pl.ds(base, ROWS_PER_TILE)],
                    out_hbm.at[pl.ds(c * NPAD + base, ROWS_PER_TILE)])

  return spmm


_spmm_cache = {}


def _spmm(*key):
  if key not in _spmm_cache:
    if key == ('bf16',):
      _spmm_cache[key] = _make_spmm_bf16()
    else:
      _spmm_cache[key] = _make_spmm(*key)
  return _spmm_cache[key]


def _bn(t, g, b):
  m = jnp.mean(t, axis=0, keepdims=True)
  v = jnp.mean(jnp.square(t - m), axis=0, keepdims=True)
  return g * (t - m) / jnp.sqrt(v + 1e-5) + b


def _psum(p_ref):
  return p_ref[pl.ds(0, N), :] + p_ref[pl.ds(NPAD, N), :]


def _tc_mid_body(p_ref, w_ref, g_ref, b_ref, tra_ref, o_ref):
  t = jnp.dot(_psum(p_ref), w_ref[...], preferred_element_type=jnp.float32)
  h = jnp.maximum(_bn(t, g_ref[...], b_ref[...]), 0.0)
  o_ref[...] = ((1.0 - SIGMA) * h + SIGMA * tra_ref[...]).astype(jnp.bfloat16)


def _tc_mid(p, w, g, b, tra):
  return pl.pallas_call(
      _tc_mid_body,
      out_shape=jax.ShapeDtypeStruct((N, 128), jnp.bfloat16),
  )(p, w, g, b, tra)


def _tc_l3_body(p_ref, w_ref, g_ref, b_ref, tra_ref, w4_ref, o_ref):
  t = jnp.dot(_psum(p_ref), w_ref[...], preferred_element_type=jnp.float32)
  h = jnp.maximum(_bn(t, g_ref[...], b_ref[...]), 0.0)
  u = (1.0 - SIGMA) * h + SIGMA * tra_ref[...]
  o_ref[...] = jnp.dot(u, w4_ref[...], preferred_element_type=jnp.float32)


def _tc_l3(p, w, g, b, tra, w4p):
  return pl.pallas_call(
      _tc_l3_body,
      out_shape=jax.ShapeDtypeStruct((N, 16), jnp.float32),
  )(p, w, g, b, tra, w4p)


def _psum16(p_ref):
  return (p_ref[pl.ds(0, N), :] + p_ref[pl.ds(NPAD, N), :])[:, :10]


def _tc_l4_body(p_ref, g_ref, b_ref, z_ref, w5_ref, o_ref):
  h = jnp.maximum(_bn(_psum16(p_ref), g_ref[...], b_ref[...]), 0.0)
  u = (1.0 - SIGMA) * h + SIGMA * z_ref[...]
  o_ref[...] = jnp.dot(u, w5_ref[...], preferred_element_type=jnp.float32)


def _tc_l4(p, g, b, z, w5p):
  return pl.pallas_call(
      _tc_l4_body,
      out_shape=jax.ShapeDtypeStruct((N, 16), jnp.float32),
  )(p, g, b, z, w5p)


def _tc_final_body(p_ref, g_ref, b_ref, z_ref, cl_ref,
                   q_ref, pred_ref, h5_ref):
  h5 = _bn(_psum16(p_ref), g_ref[...], b_ref[...])
  h5_ref[...] = h5
  nrm = jnp.sqrt(jnp.sum(h5 * h5, axis=1, keepdims=True)) + 1e-12
  pred_ref[...] = jax.nn.softmax(h5 / nrm, axis=1)
  z = z_ref[...]
  cl = cl_ref[...]
  z2 = jnp.sum(z * z, axis=1, keepdims=True)
  c2 = jnp.sum(cl * cl, axis=1)
  zc = lax.dot_general(z, cl, (((1,), (1,)), ((), ())),
                       preferred_element_type=jnp.float32)
  d2 = z2 + c2[None, :] - 2.0 * zc
  q = 1.0 / (1.0 + d2)
  q_ref[...] = q / jnp.sum(q, axis=1, keepdims=True)


def _tc_final(p, g, b, z, cl):
  return pl.pallas_call(
      _tc_final_body,
      out_shape=(
          jax.ShapeDtypeStruct((N, 4), jnp.float32),
          jax.ShapeDtypeStruct((N, 10), jnp.float32),
          jax.ShapeDtypeStruct((N, 10), jnp.float32),
      ),
  )(p, g, b, z, cl)


_PERM = np.array(
    [32 * g + (i // 2 if i % 2 == 0 else 16 + i // 2)
     for g in range(4) for i in range(32)])


def _pack_bf16(xbf):
  return jax.lax.bitcast_convert_type(
      xbf.reshape(xbf.shape[0], 64, 2), jnp.int32)


def kernel(encoded_input_data, tra1, tra2, tra3, z, edge_index, edge_weight,
           W1, g1, b1, W2, g2, b2, W3, g3, b3, W4, g4, b4, W5, g5, b5,
           cluster):
  pad = EPAD - E
  src = jnp.concatenate([edge_index[0], jnp.zeros((pad,), jnp.int32)])
  dst = jnp.concatenate([edge_index[1], jnp.zeros((pad,), jnp.int32)])
  wgt = jnp.concatenate([edge_weight, jnp.zeros((pad,), jnp.float32)])
  src64 = src.reshape(EPAD // 64, 64)
  src128 = src.reshape(EPAD // 128, 128)

  w4p = jnp.pad(W4, ((0, 0), (0, 6)))
  w5p = jnp.pad(W5, ((0, 0), (0, 6)))

  spmm128 = _spmm('bf16')
  spmm16 = _spmm(16, 128, False)

  perm = _PERM
  xp = _pack_bf16(encoded_input_data[:, perm].astype(jnp.bfloat16))
  p1 = spmm128(xp, src64, dst, wgt)
  u2 = _tc_mid(p1, W1[:, perm], g1[perm], b1[perm], tra1[:, perm])
  p2 = spmm128(_pack_bf16(u2), src64, dst, wgt)
  u3 = _tc_mid(p2, W2[:, perm], g2[perm], b2[perm], tra2[:, perm])
  p3 = spmm128(_pack_bf16(u3), src64, dst, wgt)
  y4 = _tc_l3(p3, W3, g3, b3, tra3, w4p)
  p4 = spmm16(y4, src128, dst, wgt)
  y5 = _tc_l4(p4, g4, b4, z, w5p)
  p5 = spmm16(y5, src128, dst, wgt)
  q, pred, h5 = _tc_final(p5, g5, b5, z, cluster)
  return (q, pred, h5)


# speedup vs baseline: 1.5275x; 1.0064x over previous
import functools

import numpy as np

import jax
import jax.numpy as jnp
from jax import lax
from jax.experimental import pallas as pl
from jax.experimental.pallas import tpu as pltpu
from jax.experimental.pallas import tpu_sc as plsc

N = 10000
NPAD = 10240
E = 320000
EPAD = 327680
ROWS_PER_TILE = NPAD // 16
SIGMA = 0.5
NB = 4


def _make_spmm(dsc, ce, col_split):
  nchunks = EPAD // ce
  nstage = nchunks // 16 if col_split else nchunks // 32
  mesh = plsc.VectorSubcoreMesh(core_axis_name="c", subcore_axis_name="s",
                                num_cores=2, num_subcores=16)

  @functools.partial(
      pl.kernel,
      out_type=jax.ShapeDtypeStruct((2 * NPAD, dsc), jnp.float32),
      mesh=mesh,
      scratch_types=[
          pltpu.VMEM((nstage, ce), jnp.int32),
          [pltpu.VMEM((ce, dsc), jnp.float32)] * NB,
          [pltpu.VMEM((ce,), jnp.float32)] * NB,
          [pltpu.VMEM((ce,), jnp.int32)] * NB,
          pltpu.VMEM_SHARED((NPAD, dsc), jnp.float32),
          [pltpu.SemaphoreType.DMA] * NB,
          [pltpu.SemaphoreType.DMA] * NB,
      ],
      compiler_params=pltpu.CompilerParams(use_tc_tiling_on_sc=False),
  )
  def spmm(x_hbm, src_hbm, dst_hbm, w_hbm, out_hbm,
           src_v, bufs, wslots, dslots, acc_s, gsems, ssems):
    c = lax.axis_index("c")
    s = lax.axis_index("s")
    ebase = (s * 2 + c) * nstage
    base = s * ROWS_PER_TILE

    r0 = bufs[0]
    zero16 = jnp.zeros((16,), jnp.float32)

    def zrow(i, carry):
      for cb in range(dsc // 16):
        r0[i, pl.ds(cb * 16, 16)] = zero16
      return carry

    lax.fori_loop(0, ce, zrow, 0, unroll=2)

    for r in range(ROWS_PER_TILE // ce):
      pltpu.sync_copy(r0, acc_s.at[pl.ds(base + r * ce, ce)])

    pltpu.sync_copy(src_hbm.at[pl.ds(ebase, nstage)], src_v)

    plsc.subcore_barrier()

    def gather(j, b):
      pltpu.async_copy(x_hbm.at[src_v.at[j]], bufs[b], gsems[b])
      off = (ebase + j) * ce
      pltpu.async_copy(w_hbm.at[pl.ds(off, ce)], wslots[b], gsems[b])
      pltpu.async_copy(dst_hbm.at[pl.ds(off, ce)], dslots[b], gsems[b])

    def gwait(b):
      pltpu.make_async_copy(x_hbm.at[pl.ds(0, ce)], bufs[b], gsems[b]).wait()
      pltpu.make_async_copy(w_hbm.at[pl.ds(0, ce)], wslots[b], gsems[b]).wait()
      pltpu.make_async_copy(dst_hbm.at[pl.ds(0, ce)], dslots[b],
                            gsems[b]).wait()

    def scatter(j, b):
      pltpu.async_copy(bufs[b], acc_s.at[dslots[b]], ssems[b], add=True)

    def swait(b):
      pltpu.make_async_copy(x_hbm.at[pl.ds(0, ce)], bufs[b], ssems[b]).wait()

    def mult(j, b):
      buf = bufs[b]
      wsl = wslots[b]

      def grp_body(g, cc):
        wvec = wsl[pl.ds(g * 16, 16)]
        for l in range(16):
          wl = wvec[l]
          e = g * 16 + l
          for cb in range(dsc // 16):
            sl = pl.ds(cb * 16, 16)
            buf[e, sl] = buf[e, sl] * wl
        return cc

      lax.fori_loop(0, ce // 16, grp_body, 0)

    gather(0, 0)
    gather(1, 1)
    for b in range(NB):
      gwait(b)
      mult(b, b)
      scatter(b, b)
      b2 = (b + 2) % NB
      if b >= NB - 2:
        swait(b2)
      gather(b + 2, b2)

    def block_body(g, carry):
      for b in range(NB):
        j = g * NB + b
        gwait(b)
        mult(j, b)
        scatter(j, b)
        b2 = (b + 2) % NB
        swait(b2)
        gather(j + 2, b2)
      return carry

    lax.fori_loop(1, nstage // NB - 1, block_body, 0)

    for b in range(NB):
      j = nstage - NB + b
      gwait(b)
      mult(j, b)
      scatter(j, b)
      if b < NB - 2:
        b2 = (b + 2) % NB
        swait(b2)
        gather(j + 2, b2)
    for b in range(NB):
      swait(b)

    plsc.subcore_barrier()

    pltpu.sync_copy(acc_s.at[pl.ds(base, ROWS_PER_TILE)],
                    out_hbm.at[pl.ds(c * NPAD + base, ROWS_PER_TILE)])

  return spmm


def _make_spmm_bf16():
  ce = 64
  nstage = EPAD // ce // 32
  mesh = plsc.VectorSubcoreMesh(core_axis_name="c", subcore_axis_name="s",
                                num_cores=2, num_subcores=16)

  @functools.partial(
      pl.kernel,
      out_type=jax.ShapeDtypeStruct((2 * NPAD, 128), jnp.float32),
      mesh=mesh,
      scratch_types=[
          pltpu.VMEM((nstage, ce), jnp.int32),
          [pltpu.VMEM((ce, 64), jnp.int32)] * 4,
          [pltpu.VMEM((ce, 128), jnp.float32)] * 2,
          [pltpu.VMEM((ce,), jnp.float32)] * 4,
          [pltpu.VMEM((ce,), jnp.int32)] * 4,
          pltpu.VMEM_SHARED((NPAD, 128), jnp.float32),
          [pltpu.SemaphoreType.DMA] * 4,
          [pltpu.SemaphoreType.DMA] * 4,
          [pltpu.SemaphoreType.DMA] * 2,
      ],
      compiler_params=pltpu.CompilerParams(use_tc_tiling_on_sc=False,
                                     needs_layout_passes=False),
  )
  def spmm(x_hbm, src_hbm, dst_hbm, w_hbm, out_hbm,
           src_v, gbufs, fbufs, wslots, dslots, acc_s, gsems, esems, ssems):
    c = lax.axis_index("c")
    s = lax.axis_index("s")
    ebase = (s * 2 + c) * nstage
    base = s * ROWS_PER_TILE

    f0 = fbufs[0]
    zero16 = jnp.zeros((16,), jnp.float32)

    def zrow(i, carry):
      for cb in range(8):
        f0[i, pl.ds(cb * 16, 16)] = zero16
      return carry

    lax.fori_loop(0, ce, zrow, 0, unroll=2)

    for r in range(ROWS_PER_TILE // ce):
      pltpu.sync_copy(f0, acc_s.at[pl.ds(base + r * ce, ce)])

    pltpu.sync_copy(src_hbm.at[pl.ds(ebase, nstage)], src_v)

    plsc.subcore_barrier()

    def gather(j, b):
      pltpu.async_copy(x_hbm.at[src_v.at[j]], gbufs[b], gsems[b])

    def gwait(b):
      pltpu.make_async_copy(src_hbm.at[pl.ds(0, ce)], gbufs[b],
                            gsems[b]).wait()

    def load_wd(j, b):
      off = (ebase + j) * ce
      pltpu.async_copy(w_hbm.at[pl.ds(off, ce)], wslots[b], esems[b])
      pltpu.async_copy(dst_hbm.at[pl.ds(off, ce)], dslots[b], esems[b])

    def ewait(b):
      pltpu.make_async_copy(w_hbm.at[pl.ds(0, ce)], wslots[b],
                            esems[b]).wait()
      pltpu.make_async_copy(dst_hbm.at[pl.ds(0, ce)], dslots[b],
                            esems[b]).wait()

    def scatter(j, b4, b2):
      pltpu.async_copy(fbufs[b2], acc_s.at[dslots[b4]], ssems[b2], add=True)

    def swait(b2):
      pltpu.make_async_copy(out_hbm.at[pl.ds(0, ce)], fbufs[b2],
                            ssems[b2]).wait()

    himask = jnp.int32(-65536)

    def mult(j, b4, b2):
      gb = gbufs[b4]
      fbuf = fbufs[b2]
      wsl = wslots[b4]

      def grp_body(g, cc):
        wvec = wsl[pl.ds(g * 16, 16)]
        for l in range(16):
          wl = wvec[l]
          e = g * 16 + l
          for k in range(4):
            w32 = gb[e, pl.ds(k * 16, 16)]
            lo = plsc.bitcast(w32 << 16, jnp.float32)
            hi = plsc.bitcast(w32 & himask, jnp.float32)
            fbuf[e, pl.ds(k * 32, 16)] = lo * wl
            fbuf[e, pl.ds(k * 32 + 16, 16)] = hi * wl
        return cc

      lax.fori_loop(0, ce // 16, grp_body, 0)

    def body(j, b4, b2, with_gather, with_swait, with_wd):
      gwait(b4)
      if with_swait:
        swait(b2)
      if with_wd:
        load_wd(j + 2, (b4 + 2) % 4)
      ewait(b4)
      mult(j, b4, b2)
      scatter(j, b4, b2)
      if with_gather:
        gather(j + 4, b4)

    gather(0, 0)
    gather(1, 1)
    gather(2, 2)
    gather(3, 3)
    load_wd(0, 0)
    load_wd(1, 1)
    for j in range(4):
      body(j, j % 4, j % 2, True, j >= 2, True)

    def block_body(g, carry):
      for b in range(4):
        body(g * 4 + b, b, b % 2, True, True, True)
      return carry

    lax.fori_loop(1, nstage // 4 - 1, block_body, 0)

    for b in range(4):
      j = nstage - 4 + b
      body(j, b, b % 2, j + 4 < nstage, True, j + 2 < nstage)
    swait(0)
    swait(1)

    plsc.subcore_barrier()

    pltpu.sync_copy(acc_s.at[pl.ds(base, ROWS_PER_TILE)],
                    out_hbm.at[pl.ds(c * NPAD + base, ROWS_PER_TILE)])

  return spmm


_spmm_cache = {}


def _spmm(*key):
  if key not in _spmm_cache:
    if key == ('bf16',):
      _spmm_cache[key] = _make_spmm_bf16()
    else:
      _spmm_cache[key] = _make_spmm(*key)
  return _spmm_cache[key]


def _bn(t, g, b):
  m = jnp.mean(t, axis=0, keepdims=True)
  v = jnp.mean(jnp.square(t - m), axis=0, keepdims=True)
  return g * (t - m) / jnp.sqrt(v + 1e-5) + b


def _psum(p_ref):
  return p_ref[pl.ds(0, N), :] + p_ref[pl.ds(NPAD, N), :]


def _tc_mid_body(p_ref, w_ref, g_ref, b_ref, tra_ref, o_ref):
  t = jnp.dot(_psum(p_ref), w_ref[...], preferred_element_type=jnp.float32)
  h = jnp.maximum(_bn(t, g_ref[...], b_ref[...]), 0.0)
  o_ref[...] = ((1.0 - SIGMA) * h + SIGMA * tra_ref[...]).astype(jnp.bfloat16)


def _tc_mid(p, w, g, b, tra):
  return pl.pallas_call(
      _tc_mid_body,
      out_shape=jax.ShapeDtypeStruct((N, 128), jnp.bfloat16),
  )(p, w, g, b, tra)


def _tc_l3_body(p_ref, w_ref, g_ref, b_ref, tra_ref, w4_ref, o_ref):
  t = jnp.dot(_psum(p_ref), w_ref[...], preferred_element_type=jnp.float32)
  h = jnp.maximum(_bn(t, g_ref[...], b_ref[...]), 0.0)
  u = (1.0 - SIGMA) * h + SIGMA * tra_ref[...]
  o_ref[...] = jnp.dot(u, w4_ref[...], preferred_element_type=jnp.float32)


def _tc_l3(p, w, g, b, tra, w4p):
  return pl.pallas_call(
      _tc_l3_body,
      out_shape=jax.ShapeDtypeStruct((N, 16), jnp.float32),
  )(p, w, g, b, tra, w4p)


def _psum16(p_ref):
  return (p_ref[pl.ds(0, N), :] + p_ref[pl.ds(NPAD, N), :])[:, :10]


def _tc_l4_body(p_ref, g_ref, b_ref, z_ref, w5_ref, o_ref):
  h = jnp.maximum(_bn(_psum16(p_ref), g_ref[...], b_ref[...]), 0.0)
  u = (1.0 - SIGMA) * h + SIGMA * z_ref[...]
  o_ref[...] = jnp.dot(u, w5_ref[...], preferred_element_type=jnp.float32)


def _tc_l4(p, g, b, z, w5p):
  return pl.pallas_call(
      _tc_l4_body,
      out_shape=jax.ShapeDtypeStruct((N, 16), jnp.float32),
  )(p, g, b, z, w5p)


def _tc_final_body(p_ref, g_ref, b_ref, z_ref, cl_ref,
                   q_ref, pred_ref, h5_ref):
  h5 = _bn(_psum16(p_ref), g_ref[...], b_ref[...])
  h5_ref[...] = h5
  nrm = jnp.sqrt(jnp.sum(h5 * h5, axis=1, keepdims=True)) + 1e-12
  pred_ref[...] = jax.nn.softmax(h5 / nrm, axis=1)
  z = z_ref[...]
  cl = cl_ref[...]
  z2 = jnp.sum(z * z, axis=1, keepdims=True)
  c2 = jnp.sum(cl * cl, axis=1)
  zc = lax.dot_general(z, cl, (((1,), (1,)), ((), ())),
                       preferred_element_type=jnp.float32)
  d2 = z2 + c2[None, :] - 2.0 * zc
  q = 1.0 / (1.0 + d2)
  q_ref[...] = q / jnp.sum(q, axis=1, keepdims=True)


def _tc_final(p, g, b, z, cl):
  return pl.pallas_call(
      _tc_final_body,
      out_shape=(
          jax.ShapeDtypeStruct((N, 4), jnp.float32),
          jax.ShapeDtypeStruct((N, 10), jnp.float32),
          jax.ShapeDtypeStruct((N, 10), jnp.float32),
      ),
  )(p, g, b, z, cl)


_PERM = np.array(
    [32 * g + (i // 2 if i % 2 == 0 else 16 + i // 2)
     for g in range(4) for i in range(32)])


def _pack_bf16(xbf):
  return jax.lax.bitcast_convert_type(
      xbf.reshape(xbf.shape[0], 64, 2), jnp.int32)


def kernel(encoded_input_data, tra1, tra2, tra3, z, edge_index, edge_weight,
           W1, g1, b1, W2, g2, b2, W3, g3, b3, W4, g4, b4, W5, g5, b5,
           cluster):
  pad = EPAD - E
  src = jnp.concatenate([edge_index[0], jnp.zeros((pad,), jnp.int32)])
  dst = jnp.concatenate([edge_index[1], jnp.zeros((pad,), jnp.int32)])
  wgt = jnp.concatenate([edge_weight, jnp.zeros((pad,), jnp.float32)])
  src64 = src.reshape(EPAD // 64, 64)
  src128 = src.reshape(EPAD // 128, 128)

  w4p = jnp.pad(W4, ((0, 0), (0, 6)))
  w5p = jnp.pad(W5, ((0, 0), (0, 6)))

  spmm128 = _spmm('bf16')
  spmm16 = _spmm(16, 128, False)

  perm = _PERM
  xp = _pack_bf16(encoded_input_data[:, perm].astype(jnp.bfloat16))
  p1 = spmm128(xp, src64, dst, wgt)
  u2 = _tc_mid(p1, W1[:, perm], g1[perm], b1[perm], tra1[:, perm])
  p2 = spmm128(_pack_bf16(u2), src64, dst, wgt)
  u3 = _tc_mid(p2, W2[:, perm], g2[perm], b2[perm], tra2[:, perm])
  p3 = spmm128(_pack_bf16(u3), src64, dst, wgt)
  y4 = _tc_l3(p3, W3, g3, b3, tra3, w4p)
  p4 = spmm16(y4, src128, dst, wgt)
  y5 = _tc_l4(p4, g4, b4, z, w5p)
  p5 = spmm16(y5, src128, dst, wgt)
  q, pred, h5 = _tc_final(p5, g5, b5, z, cluster)
  return (q, pred, h5)
